# baseline (speedup 1.0000x reference)
import functools

import jax
import jax.numpy as jnp
from jax import lax
from jax.experimental import pallas as pl
from jax.experimental.pallas import tpu as pltpu
from jax.experimental.pallas import tpu_sc as plsc

_LANES = 16
_CHUNK = 128


@functools.partial(jax.jit, static_argnums=())
def kernel(user_ix, item_ix, user_emb, item_emb):
    B = user_ix.shape[0]
    F = user_emb.shape[1]
    info = plsc.get_sparse_core_info()
    NC, NS = info.num_cores, info.num_subcores
    NW = NC * NS
    b_per_w = B // NW
    k = b_per_w // _CHUNK
    assert B == NW * b_per_w and b_per_w == k * _CHUNK and F % _LANES == 0

    uix = user_ix.reshape(NW, k, _CHUNK)
    iix = item_ix.reshape(NW, k, _CHUNK)

    mesh = plsc.VectorSubcoreMesh(core_axis_name="c", subcore_axis_name="s")

    @functools.partial(
        pl.kernel,
        mesh=mesh,
        out_type=jax.ShapeDtypeStruct((B, F), jnp.float32),
        compiler_params=pltpu.CompilerParams(use_tc_tiling_on_sc=False),
        scratch_types=[
            pltpu.VMEM((k, _CHUNK), jnp.int32),
            pltpu.VMEM((k, _CHUNK), jnp.int32),
            pltpu.VMEM((b_per_w, F), jnp.float32),
            pltpu.VMEM((b_per_w, F), jnp.float32),
            pltpu.SemaphoreType.DMA,
        ],
    )
    def run(uix_hbm, iix_hbm, uemb_hbm, iemb_hbm, out_hbm,
            uidx_v, iidx_v, urows_v, irows_v, sem):
        wid = lax.axis_index("s") * NC + lax.axis_index("c")
        base = wid * b_per_w
        pltpu.sync_copy(uix_hbm.at[wid], uidx_v)
        pltpu.sync_copy(iix_hbm.at[wid], iidx_v)
        copies = []
        for j in range(k):
            dst = pl.ds(j * _CHUNK, _CHUNK)
            copies.append(
                pltpu.async_copy(uemb_hbm.at[uidx_v.at[j]], urows_v.at[dst], sem))
            copies.append(
                pltpu.async_copy(iemb_hbm.at[iidx_v.at[j]], irows_v.at[dst], sem))
        for c in copies:
            c.wait()

        nvec = F // _LANES

        def mul_row(r, carry):
            for c in range(nvec):
                sl = pl.ds(c * _LANES, _LANES)
                urows_v[r, sl] = urows_v[r, sl] * irows_v[r, sl]
            return carry

        lax.fori_loop(0, b_per_w, mul_row, 0)
        pltpu.sync_copy(urows_v, out_hbm.at[pl.ds(base, b_per_w)])

    return run(uix, iix, user_emb, item_emb)


# speedup vs baseline: 6.1476x; 6.1476x over previous
import functools

import jax
import jax.numpy as jnp
from jax import lax
from jax.experimental import pallas as pl
from jax.experimental.pallas import tpu as pltpu
from jax.experimental.pallas import tpu_sc as plsc

_LANES = 16


def kernel(user_ix, item_ix, user_emb, item_emb):
    B = user_ix.shape[0]
    V, F = user_emb.shape
    info = plsc.get_sparse_core_info()
    NC, NS = info.num_cores, info.num_subcores
    NW = NC * NS
    C = B // NW
    G = F // 8
    nblk = C // _LANES
    assert B == NW * C and F == 8 * G and C == nblk * _LANES

    ut3 = user_emb.T.reshape(G, 8, V)
    it3 = item_emb.T.reshape(G, 8, V)
    uixf = user_ix.reshape(B)
    iixf = item_ix.reshape(B)

    mesh = plsc.VectorSubcoreMesh(core_axis_name="c", subcore_axis_name="s")

    @functools.partial(
        pl.kernel,
        mesh=mesh,
        out_type=jax.ShapeDtypeStruct((F, B), jnp.float32),
        compiler_params=pltpu.CompilerParams(needs_layout_passes=False),
        scratch_types=[
            pltpu.VMEM((C,), jnp.int32),
            pltpu.VMEM((C,), jnp.int32),
            pltpu.VMEM((G, 8, _LANES * _LANES), jnp.float32),
            pltpu.VMEM((G, 8, _LANES * _LANES), jnp.float32),
            pltpu.VMEM((G, 8, C), jnp.float32),
            pltpu.SemaphoreType.DMA,
            pltpu.SemaphoreType.DMA,
        ],
    )
    def run(ut_hbm, it_hbm, uix_hbm, iix_hbm, out_hbm,
            uidx_v, iidx_v, ublk_v, iblk_v, prod_v, sem_u, sem_i):
        wid = lax.axis_index("s") * NC + lax.axis_index("c")
        base = wid * C
        pltpu.sync_copy(uix_hbm.at[pl.ds(base, C)], uidx_v)
        pltpu.sync_copy(iix_hbm.at[pl.ds(base, C)], iidx_v)

        iota = lax.iota(jnp.int32, _LANES)

        def block(bb, _):
            sl = pl.ds(bb * _LANES, _LANES)
            uvec = uidx_v[sl]
            ivec = iidx_v[sl]
            ual = uvec & jnp.int32(-_LANES)
            ial = ivec & jnp.int32(-_LANES)
            for j in range(_LANES):
                dst = pl.ds(j * _LANES, _LANES)
                ub = pl.multiple_of(ual[j], _LANES)
                ib = pl.multiple_of(ial[j], _LANES)
                pltpu.async_copy(
                    ut_hbm.at[:, :, pl.ds(ub, _LANES)],
                    ublk_v.at[:, :, dst], sem_u)
                pltpu.async_copy(
                    it_hbm.at[:, :, pl.ds(ib, _LANES)],
                    iblk_v.at[:, :, dst], sem_i)
            pltpu.make_async_copy(
                ut_hbm.at[:, :, pl.ds(0, _LANES * _LANES)], ublk_v,
                sem_u).wait()
            pltpu.make_async_copy(
                it_hbm.at[:, :, pl.ds(0, _LANES * _LANES)], iblk_v,
                sem_i).wait()

            ulane = iota * _LANES + (uvec & (_LANES - 1))
            ilane = iota * _LANES + (ivec & (_LANES - 1))
            for g in range(G):
                gg = jnp.full((_LANES,), g, jnp.int32)
                for s in range(8):
                    ss = jnp.full((_LANES,), s, jnp.int32)
                    u16 = plsc.load_gather(ublk_v, [gg, ss, ulane])
                    i16 = plsc.load_gather(iblk_v, [gg, ss, ilane])
                    prod_v[g, s, sl] = u16 * i16
            return 0

        lax.fori_loop(0, nblk, block, 0)

        for g in range(G):
            for s in range(8):
                pltpu.sync_copy(
                    prod_v.at[g, s], out_hbm.at[8 * g + s, pl.ds(base, C)])

    out = run(ut3, it3, uixf, iixf)
    return out.T


# speedup vs baseline: 6.2628x; 1.0187x over previous
import functools

import jax
import jax.numpy as jnp
from jax import lax
from jax.experimental import pallas as pl
from jax.experimental.pallas import tpu as pltpu
from jax.experimental.pallas import tpu_sc as plsc

_LANES = 16


def kernel(user_ix, item_ix, user_emb, item_emb):
    B = user_ix.shape[0]
    V, F = user_emb.shape
    info = plsc.get_sparse_core_info()
    NC, NS = info.num_cores, info.num_subcores
    NW = NC * NS
    C = B // NW
    G = F // 8
    nblk = C // _LANES
    assert B == NW * C and F == 8 * G and C == nblk * _LANES and nblk % 2 == 0

    ut3 = user_emb.T.reshape(G, 8, V)
    it3 = item_emb.T.reshape(G, 8, V)
    uixf = user_ix.reshape(B)
    iixf = item_ix.reshape(B)

    mesh = plsc.VectorSubcoreMesh(core_axis_name="c", subcore_axis_name="s")

    BLK = _LANES * _LANES

    @functools.partial(
        pl.kernel,
        mesh=mesh,
        out_type=jax.ShapeDtypeStruct((F, B), jnp.float32),
        compiler_params=pltpu.CompilerParams(needs_layout_passes=False),
        scratch_types=[
            pltpu.VMEM((C,), jnp.int32),
            pltpu.VMEM((C,), jnp.int32),
            pltpu.VMEM((G, 8, BLK), jnp.float32),
            pltpu.VMEM((G, 8, BLK), jnp.float32),
            pltpu.VMEM((G, 8, BLK), jnp.float32),
            pltpu.VMEM((G, 8, BLK), jnp.float32),
            pltpu.VMEM((G, 8, C), jnp.float32),
            pltpu.SemaphoreType.DMA,
            pltpu.SemaphoreType.DMA,
            pltpu.SemaphoreType.DMA,
            pltpu.SemaphoreType.DMA,
        ],
    )
    def run(ut_hbm, it_hbm, uix_hbm, iix_hbm, out_hbm,
            uidx_v, iidx_v, ublk0, iblk0, ublk1, iblk1, prod_v,
            sem_u0, sem_i0, sem_u1, sem_i1):
        wid = lax.axis_index("s") * NC + lax.axis_index("c")
        base = wid * C
        pltpu.sync_copy(uix_hbm.at[pl.ds(base, C)], uidx_v)
        pltpu.sync_copy(iix_hbm.at[pl.ds(base, C)], iidx_v)

        iota = lax.iota(jnp.int32, _LANES)

        def fire(bb, ublk, iblk, su, si):
            sl = pl.ds(bb * _LANES, _LANES)
            ual = uidx_v[sl] & jnp.int32(-_LANES)
            ial = iidx_v[sl] & jnp.int32(-_LANES)
            for j in range(_LANES):
                dst = pl.ds(j * _LANES, _LANES)
                ub = pl.multiple_of(ual[j], _LANES)
                ib = pl.multiple_of(ial[j], _LANES)
                pltpu.async_copy(
                    ut_hbm.at[:, :, pl.ds(ub, _LANES)], ublk.at[:, :, dst], su)
                pltpu.async_copy(
                    it_hbm.at[:, :, pl.ds(ib, _LANES)], iblk.at[:, :, dst], si)

        def drain_extract(bb, ublk, iblk, su, si):
            pltpu.make_async_copy(
                ut_hbm.at[:, :, pl.ds(0, BLK)], ublk, su).wait()
            pltpu.make_async_copy(
                it_hbm.at[:, :, pl.ds(0, BLK)], iblk, si).wait()
            sl = pl.ds(bb * _LANES, _LANES)
            ulane = iota * _LANES + (uidx_v[sl] & (_LANES - 1))
            ilane = iota * _LANES + (iidx_v[sl] & (_LANES - 1))
            for g in range(G):
                gg = jnp.full((_LANES,), g, jnp.int32)
                for s in range(8):
                    ss = jnp.full((_LANES,), s, jnp.int32)
                    u16 = plsc.load_gather(ublk, [gg, ss, ulane])
                    i16 = plsc.load_gather(iblk, [gg, ss, ilane])
                    prod_v[g, s, sl] = u16 * i16

        fire(0, ublk0, iblk0, sem_u0, sem_i0)

        def pair(pp, _):
            e = 2 * pp
            fire(e + 1, ublk1, iblk1, sem_u1, sem_i1)
            drain_extract(e, ublk0, iblk0, sem_u0, sem_i0)

            @pl.when(pp < nblk // 2 - 1)
            def _():
                fire(e + 2, ublk0, iblk0, sem_u0, sem_i0)

            drain_extract(e + 1, ublk1, iblk1, sem_u1, sem_i1)
            return 0

        lax.fori_loop(0, nblk // 2, pair, 0)

        for g in range(G):
            for s in range(8):
                pltpu.sync_copy(
                    prod_v.at[g, s], out_hbm.at[8 * g + s, pl.ds(base, C)])

    out = run(ut3, it3, uixf, iixf)
    return out.T


# speedup vs baseline: 6.3620x; 1.0158x over previous
import functools

import jax
import jax.numpy as jnp
from jax import lax
from jax.experimental import pallas as pl
from jax.experimental.pallas import tpu as pltpu
from jax.experimental.pallas import tpu_sc as plsc

_LANES = 16


def kernel(user_ix, item_ix, user_emb, item_emb):
    B = user_ix.shape[0]
    V, F = user_emb.shape
    info = plsc.get_sparse_core_info()
    NC, NS = info.num_cores, info.num_subcores
    NW = NC * NS
    C = B // NW
    G = F // 8
    nblk = C // _LANES
    assert B == NW * C and F == 8 * G and C == nblk * _LANES and nblk % 2 == 0

    ut3 = user_emb.T.reshape(G, 8, V)
    it3 = item_emb.T.reshape(G, 8, V)
    uixf = user_ix.reshape(B)
    iixf = item_ix.reshape(B)

    mesh = plsc.VectorSubcoreMesh(core_axis_name="c", subcore_axis_name="s")

    BLK = _LANES * _LANES

    @functools.partial(
        pl.kernel,
        mesh=mesh,
        out_type=jax.ShapeDtypeStruct((F, B), jnp.float32),
        compiler_params=pltpu.CompilerParams(needs_layout_passes=False),
        scratch_types=[
            pltpu.VMEM((C,), jnp.int32),
            pltpu.VMEM((C,), jnp.int32),
            pltpu.VMEM((G, 8, BLK), jnp.float32),
            pltpu.VMEM((G, 8, BLK), jnp.float32),
            pltpu.VMEM((G, 8, BLK), jnp.float32),
            pltpu.VMEM((G, 8, BLK), jnp.float32),
            pltpu.VMEM((G, 8, C), jnp.float32),
            pltpu.SemaphoreType.DMA,
            pltpu.SemaphoreType.DMA,
            pltpu.SemaphoreType.DMA,
            pltpu.SemaphoreType.DMA,
        ],
    )
    def run(ut_hbm, it_hbm, uix_hbm, iix_hbm, out_hbm,
            uidx_v, iidx_v, ublk0, iblk0, ublk1, iblk1, prod_v,
            sem_u0, sem_i0, sem_u1, sem_i1):
        wid = lax.axis_index("s") * NC + lax.axis_index("c")
        base = wid * C
        pltpu.sync_copy(uix_hbm.at[pl.ds(base, C)], uidx_v)
        pltpu.sync_copy(iix_hbm.at[pl.ds(base, C)], iidx_v)

        iota = lax.iota(jnp.int32, _LANES)

        def fire(bb, ublk, iblk, su, si):
            sl = pl.ds(bb * _LANES, _LANES)
            ual0 = uidx_v[sl] & jnp.int32(-_LANES)
            ial0 = iidx_v[sl] & jnp.int32(-_LANES)

            def fire4(j2, carry):
                ual, ial = carry
                for jj in range(4):
                    dst = pl.ds(j2 * (4 * _LANES) + jj * _LANES, _LANES)
                    ub = pl.multiple_of(ual[jj], _LANES)
                    ib = pl.multiple_of(ial[jj], _LANES)
                    for g in range(G):
                        pltpu.async_copy(
                            ut_hbm.at[g, :, pl.ds(ub, _LANES)],
                            ublk.at[g, :, dst], su)
                        pltpu.async_copy(
                            it_hbm.at[g, :, pl.ds(ib, _LANES)],
                            iblk.at[g, :, dst], si)
                rot = ((iota + 4) & (_LANES - 1))[:, None]
                dn = lax.GatherDimensionNumbers(
                    offset_dims=(), collapsed_slice_dims=(0,),
                    start_index_map=(0,))
                gather4 = functools.partial(
                    lax.gather, dimension_numbers=dn, slice_sizes=(1,),
                    mode=lax.GatherScatterMode.PROMISE_IN_BOUNDS)
                return (gather4(ual, rot), gather4(ial, rot))

            lax.fori_loop(0, _LANES // 4, fire4, (ual0, ial0))

        def drain_extract(bb, ublk, iblk, su, si):
            pltpu.make_async_copy(
                ut_hbm.at[:, :, pl.ds(0, BLK)], ublk, su).wait()
            pltpu.make_async_copy(
                it_hbm.at[:, :, pl.ds(0, BLK)], iblk, si).wait()
            sl = pl.ds(bb * _LANES, _LANES)
            ulane = iota * _LANES + (uidx_v[sl] & (_LANES - 1))
            ilane = iota * _LANES + (iidx_v[sl] & (_LANES - 1))
            for g in range(G):
                gg = jnp.full((_LANES,), g, jnp.int32)
                for s in range(8):
                    ss = jnp.full((_LANES,), s, jnp.int32)
                    u16 = plsc.load_gather(ublk, [gg, ss, ulane])
                    i16 = plsc.load_gather(iblk, [gg, ss, ilane])
                    prod_v[g, s, sl] = u16 * i16

        def block(bb, _):
            fire(bb, ublk0, iblk0, sem_u0, sem_i0)
            drain_extract(bb, ublk0, iblk0, sem_u0, sem_i0)
            return 0

        lax.fori_loop(0, nblk, block, 0)

        for g in range(G):
            for s in range(8):
                pltpu.sync_copy(
                    prod_v.at[g, s], out_hbm.at[8 * g + s, pl.ds(base, C)])

    out = run(ut3, it3, uixf, iixf)
    return out.T
